# baseline (speedup 1.0000x reference)
import functools

import jax
import jax.numpy as jnp
from jax import lax
from jax.experimental import pallas as pl
from jax.experimental.pallas import tpu as pltpu
from jax.experimental.pallas import tpu_sc as plsc

B, N, L, R, C = 16, 8192, 512, 5, 2
T = 8
K = B - T
NW = 32
WPB = NW // K
RPW = N // WPB
CH = 64
NCHUNK = RPW // CH


def _score_kernel(x_ref, w1_ref, s_ref):
    w = w1_ref[:].reshape(1, L)
    s = jax.lax.dot_general(w, x_ref[0], (((1,), (1,)), ((), ())),
                            preferred_element_type=jnp.float32)
    s_ref[pl.ds(pl.program_id(0), 1), :] = s


def _sc_row_block(xb, w1v, sb, c):
    li = lax.broadcasted_iota(jnp.int32, (16,), 0)

    def grp_body(g, _):
        vec = jnp.zeros((16,), jnp.float32)
        for r16 in range(16):
            accs = [jnp.zeros((16,), jnp.float32) for _ in range(4)]
            for j in range(L // 16):
                xv = xb[g * 16 + r16, pl.ds(16 * j, 16)]
                accs[j % 4] = accs[j % 4] + xv * w1v[j]
            acc = (accs[0] + accs[1]) + (accs[2] + accs[3])
            for bit in (8, 4, 2, 1):
                acc = acc + acc.at[jnp.bitwise_xor(li, bit)].get(
                    mode="promise_in_bounds")
            vec = jnp.where(li == r16, acc, vec)
        sb[pl.ds(c * CH + g * 16, 16)] = vec
        return 0
    lax.fori_loop(0, CH // 16, grp_body, 0)


def _sc_score_body(x_hbm, w1_hbm, out_hbm, xb0, xb1, w1b, sb, sem0, sem1):
    cid = lax.axis_index("c")
    sid = lax.axis_index("s")
    wid = cid * 16 + sid
    kb = wid // WPB
    n_base = (wid % WPB) * RPW

    pltpu.sync_copy(w1_hbm, w1b)
    w1v = [w1b[pl.ds(16 * j, 16)] for j in range(L // 16)]

    def chunk_src(c):
        return x_hbm.at[T + kb, pl.ds(n_base + c * CH, CH)]

    pltpu.make_async_copy(chunk_src(0), xb0, sem0).start()

    def pair_body(p, _):
        c0 = 2 * p
        pltpu.make_async_copy(chunk_src(c0), xb0, sem0).wait()

        @pl.when(c0 + 1 < NCHUNK)
        def _():
            pltpu.make_async_copy(chunk_src(c0 + 1), xb1, sem1).start()
        _sc_row_block(xb0, w1v, sb, c0)

        @pl.when(c0 + 1 < NCHUNK)
        def _():
            pltpu.make_async_copy(chunk_src(c0 + 1), xb1, sem1).wait()

            @pl.when(c0 + 2 < NCHUNK)
            def _():
                pltpu.make_async_copy(chunk_src(c0 + 2), xb0, sem0).start()
            _sc_row_block(xb1, w1v, sb, c0 + 1)
        return 0

    lax.fori_loop(0, (NCHUNK + 1) // 2, pair_body, 0)
    pltpu.sync_copy(sb, out_hbm.at[kb, pl.ds(n_base, RPW)])


def _head_kernel(st_ref, ss_ref, b1_ref, Wa_ref, ba_ref, Wb_ref, bb_ref,
                 Wc_ref, bc_ref, out_ref):
    gidx = jax.lax.broadcasted_iota(jnp.int32, (T, N), 1)
    big = jnp.int32(2**30)

    def take_extreme(v, sign):
        m = (jnp.max(v, axis=1, keepdims=True) if sign > 0
             else jnp.min(v, axis=1, keepdims=True))
        fi = jnp.min(jnp.where(v == m, gidx, big), axis=1, keepdims=True)
        v2 = jnp.where(gidx == fi,
                       jnp.float32(-jnp.inf) if sign > 0 else jnp.float32(jnp.inf),
                       v)
        return m, v2

    def extremes(vals):
        maxs = []
        v = vals
        for _ in range(R):
            m, v = take_extreme(v, +1)
            maxs.append(m)
        mins = []
        v = vals
        for _ in range(R):
            m, v = take_extreme(v, -1)
            mins.append(m)
        return jnp.concatenate(mins + maxs, axis=1)

    b1 = b1_ref[0]
    cat = jnp.concatenate(
        [extremes(st_ref[...] + b1), extremes(ss_ref[...] + b1)], axis=0)
    h = jnp.dot(cat, Wa_ref[:].T, preferred_element_type=jnp.float32) + ba_ref[:]
    h = jnp.dot(h, Wb_ref[:].T, preferred_element_type=jnp.float32) + bb_ref[:]
    o = jnp.dot(h, Wc_ref[:].T, preferred_element_type=jnp.float32) + bc_ref[:]
    out_ref[...] = o[:, None, :]


@jax.jit
def _chowder(x, w1, b1, Wa, ba, Wb, bb, Wc, bc):
    s_tc = pl.pallas_call(
        _score_kernel,
        grid=(T,),
        in_specs=[
            pl.BlockSpec((1, N, L), lambda b: (b, 0, 0)),
            pl.BlockSpec((L,), lambda b: (0,)),
        ],
        out_specs=pl.BlockSpec((T, N), lambda b: (0, 0)),
        out_shape=jax.ShapeDtypeStruct((T, N), jnp.float32),
        compiler_params=pltpu.CompilerParams(
            dimension_semantics=("arbitrary",),
        ),
    )(x, w1)

    sc_score = functools.partial(
        pl.kernel,
        mesh=plsc.VectorSubcoreMesh(core_axis_name="c", subcore_axis_name="s"),
        out_type=jax.ShapeDtypeStruct((K, N), jnp.float32),
        scratch_types=[
            pltpu.VMEM((CH, L), jnp.float32),
            pltpu.VMEM((CH, L), jnp.float32),
            pltpu.VMEM((L,), jnp.float32),
            pltpu.VMEM((RPW,), jnp.float32),
            pltpu.SemaphoreType.DMA,
            pltpu.SemaphoreType.DMA,
        ],
    )(_sc_score_body)
    s_sc = sc_score(x, w1)

    out = pl.pallas_call(
        _head_kernel,
        in_specs=[
            pl.BlockSpec((T, N), lambda: (0, 0)),
            pl.BlockSpec((K, N), lambda: (0, 0)),
            pl.BlockSpec((1,), lambda: (0,)),
            pl.BlockSpec((200, 2 * R), lambda: (0, 0)),
            pl.BlockSpec((200,), lambda: (0,)),
            pl.BlockSpec((100, 200), lambda: (0, 0)),
            pl.BlockSpec((100,), lambda: (0,)),
            pl.BlockSpec((C, 100), lambda: (0, 0)),
            pl.BlockSpec((C,), lambda: (0,)),
        ],
        out_specs=pl.BlockSpec((B, 1, C), lambda: (0, 0, 0)),
        out_shape=jax.ShapeDtypeStruct((B, 1, C), jnp.float32),
    )(s_tc, s_sc, b1, Wa, ba, Wb, bb, Wc, bc)
    return out


def kernel(x, w1, b1, Wa, ba, Wb, bb, Wc, bc):
    out = _chowder(x.astype(jnp.float32), w1, b1, Wa, ba, Wb, bb, Wc, bc)
    return (out, None)


# speedup vs baseline: 1.5715x; 1.5715x over previous
import functools

import jax
import jax.numpy as jnp
from jax import lax
from jax.experimental import pallas as pl
from jax.experimental.pallas import tpu as pltpu
from jax.experimental.pallas import tpu_sc as plsc

B, N, L, R, C = 16, 8192, 512, 5, 2
T = 8
K = B - T
NW = 32
WPB = NW // K
RPW = N // WPB
CH = 64
NCHUNK = RPW // CH


def _score_kernel(x_ref, w1_ref, s_ref):
    w = w1_ref[:].reshape(1, L)
    s = jax.lax.dot_general(w, x_ref[0], (((1,), (1,)), ((), ())),
                            preferred_element_type=jnp.float32)
    s_ref[pl.ds(pl.program_id(0), 1), :] = s


def _sc_row_block(xb, w1b, sb, c):
    li = lax.broadcasted_iota(jnp.int32, (16,), 0)

    def grp_body(g, _):
        accs = [jnp.zeros((16,), jnp.float32) for _ in range(16)]
        for h in range(2):
            w1h = [w1b[pl.ds(16 * (16 * h + j), 16)] for j in range(16)]
            for r16 in range(16):
                for j in range(16):
                    xv = xb[g * 16 + r16, pl.ds(16 * (16 * h + j), 16)]
                    accs[r16] = accs[r16] + xv * w1h[j]
        vec = jnp.zeros((16,), jnp.float32)
        for r16 in range(16):
            acc = accs[r16]
            for bit in (8, 4, 2, 1):
                acc = acc + acc.at[jnp.bitwise_xor(li, bit)].get(
                    mode="promise_in_bounds")
            vec = jnp.where(li == r16, acc, vec)
        sb[pl.ds(c * CH + g * 16, 16)] = vec
        return 0
    lax.fori_loop(0, CH // 16, grp_body, 0)


def _sc_score_body(x_hbm, w1_hbm, out_hbm, xb0, xb1, w1b, sb, sem0, sem1):
    cid = lax.axis_index("c")
    sid = lax.axis_index("s")
    wid = cid * 16 + sid
    kb = wid // WPB
    n_base = (wid % WPB) * RPW

    pltpu.sync_copy(w1_hbm, w1b)

    def chunk_src(c):
        return x_hbm.at[T + kb, pl.ds(n_base + c * CH, CH)]

    pltpu.make_async_copy(chunk_src(0), xb0, sem0).start()

    def pair_body(p, _):
        c0 = 2 * p
        pltpu.make_async_copy(chunk_src(c0), xb0, sem0).wait()

        @pl.when(c0 + 1 < NCHUNK)
        def _():
            pltpu.make_async_copy(chunk_src(c0 + 1), xb1, sem1).start()
        _sc_row_block(xb0, w1b, sb, c0)

        @pl.when(c0 + 1 < NCHUNK)
        def _():
            pltpu.make_async_copy(chunk_src(c0 + 1), xb1, sem1).wait()

            @pl.when(c0 + 2 < NCHUNK)
            def _():
                pltpu.make_async_copy(chunk_src(c0 + 2), xb0, sem0).start()
            _sc_row_block(xb1, w1b, sb, c0 + 1)
        return 0

    lax.fori_loop(0, (NCHUNK + 1) // 2, pair_body, 0)
    pltpu.sync_copy(sb, out_hbm.at[kb, pl.ds(n_base, RPW)])


def _head_kernel(st_ref, ss_ref, b1_ref, Wa_ref, ba_ref, Wb_ref, bb_ref,
                 Wc_ref, bc_ref, out_ref):
    gidx = jax.lax.broadcasted_iota(jnp.int32, (T, N), 1)
    big = jnp.int32(2**30)

    def take_extreme(v, sign):
        m = (jnp.max(v, axis=1, keepdims=True) if sign > 0
             else jnp.min(v, axis=1, keepdims=True))
        fi = jnp.min(jnp.where(v == m, gidx, big), axis=1, keepdims=True)
        v2 = jnp.where(gidx == fi,
                       jnp.float32(-jnp.inf) if sign > 0 else jnp.float32(jnp.inf),
                       v)
        return m, v2

    def extremes(vals):
        maxs = []
        v = vals
        for _ in range(R):
            m, v = take_extreme(v, +1)
            maxs.append(m)
        mins = []
        v = vals
        for _ in range(R):
            m, v = take_extreme(v, -1)
            mins.append(m)
        return jnp.concatenate(mins + maxs, axis=1)

    b1 = b1_ref[0]
    cat = jnp.concatenate(
        [extremes(st_ref[...] + b1), extremes(ss_ref[...] + b1)], axis=0)
    h = jnp.dot(cat, Wa_ref[:].T, preferred_element_type=jnp.float32) + ba_ref[:]
    h = jnp.dot(h, Wb_ref[:].T, preferred_element_type=jnp.float32) + bb_ref[:]
    o = jnp.dot(h, Wc_ref[:].T, preferred_element_type=jnp.float32) + bc_ref[:]
    out_ref[...] = o[:, None, :]


@jax.jit
def _chowder(x, w1, b1, Wa, ba, Wb, bb, Wc, bc):
    s_tc = pl.pallas_call(
        _score_kernel,
        grid=(T,),
        in_specs=[
            pl.BlockSpec((1, N, L), lambda b: (b, 0, 0)),
            pl.BlockSpec((L,), lambda b: (0,)),
        ],
        out_specs=pl.BlockSpec((T, N), lambda b: (0, 0)),
        out_shape=jax.ShapeDtypeStruct((T, N), jnp.float32),
        compiler_params=pltpu.CompilerParams(
            dimension_semantics=("arbitrary",),
        ),
    )(x, w1)

    sc_score = functools.partial(
        pl.kernel,
        mesh=plsc.VectorSubcoreMesh(core_axis_name="c", subcore_axis_name="s"),
        out_type=jax.ShapeDtypeStruct((K, N), jnp.float32),
        scratch_types=[
            pltpu.VMEM((CH, L), jnp.float32),
            pltpu.VMEM((CH, L), jnp.float32),
            pltpu.VMEM((L,), jnp.float32),
            pltpu.VMEM((RPW,), jnp.float32),
            pltpu.SemaphoreType.DMA,
            pltpu.SemaphoreType.DMA,
        ],
    )(_sc_score_body)
    s_sc = sc_score(x, w1)

    out = pl.pallas_call(
        _head_kernel,
        in_specs=[
            pl.BlockSpec((T, N), lambda: (0, 0)),
            pl.BlockSpec((K, N), lambda: (0, 0)),
            pl.BlockSpec((1,), lambda: (0,)),
            pl.BlockSpec((200, 2 * R), lambda: (0, 0)),
            pl.BlockSpec((200,), lambda: (0,)),
            pl.BlockSpec((100, 200), lambda: (0, 0)),
            pl.BlockSpec((100,), lambda: (0,)),
            pl.BlockSpec((C, 100), lambda: (0, 0)),
            pl.BlockSpec((C,), lambda: (0,)),
        ],
        out_specs=pl.BlockSpec((B, 1, C), lambda: (0, 0, 0)),
        out_shape=jax.ShapeDtypeStruct((B, 1, C), jnp.float32),
    )(s_tc, s_sc, b1, Wa, ba, Wb, bb, Wc, bc)
    return out


def kernel(x, w1, b1, Wa, ba, Wb, bb, Wc, bc):
    out = _chowder(x.astype(jnp.float32), w1, b1, Wa, ba, Wb, bb, Wc, bc)
    return (out, None)


# speedup vs baseline: 2.0405x; 1.2984x over previous
import functools

import jax
import jax.numpy as jnp
from jax import lax
from jax.experimental import pallas as pl
from jax.experimental.pallas import tpu as pltpu
from jax.experimental.pallas import tpu_sc as plsc

B, N, L, R, C = 16, 8192, 512, 5, 2
T = 12
K = B - T
NW = 32
WPB = NW // K
RPW = N // WPB
CH = 64
NCHUNK = RPW // CH


def _score_kernel(x_ref, w1_ref, s_ref):
    w = w1_ref[:].reshape(1, L)
    s = jax.lax.dot_general(w, x_ref[0], (((1,), (1,)), ((), ())),
                            preferred_element_type=jnp.float32)
    s_ref[pl.ds(pl.program_id(0), 1), :] = s


def _sc_row_block(xb, w1b, sb, c):
    li = lax.broadcasted_iota(jnp.int32, (16,), 0)

    def grp_body(g, _):
        accs = [jnp.zeros((16,), jnp.float32) for _ in range(16)]
        for h in range(2):
            w1h = [w1b[pl.ds(16 * (16 * h + j), 16)] for j in range(16)]
            for r16 in range(16):
                for j in range(16):
                    xv = xb[g * 16 + r16, pl.ds(16 * (16 * h + j), 16)]
                    accs[r16] = accs[r16] + xv * w1h[j]
        vec = jnp.zeros((16,), jnp.float32)
        for r16 in range(16):
            acc = accs[r16]
            for bit in (8, 4, 2, 1):
                acc = acc + acc.at[jnp.bitwise_xor(li, bit)].get(
                    mode="promise_in_bounds")
            vec = jnp.where(li == r16, acc, vec)
        sb[pl.ds(c * CH + g * 16, 16)] = vec
        return 0
    lax.fori_loop(0, CH // 16, grp_body, 0)


def _sc_score_body(x_hbm, w1_hbm, out_hbm, xb0, xb1, w1b, sb, sem0, sem1):
    cid = lax.axis_index("c")
    sid = lax.axis_index("s")
    wid = cid * 16 + sid
    kb = wid // WPB
    n_base = (wid % WPB) * RPW

    pltpu.sync_copy(w1_hbm, w1b)

    def chunk_src(c):
        return x_hbm.at[T + kb, pl.ds(n_base + c * CH, CH)]

    pltpu.make_async_copy(chunk_src(0), xb0, sem0).start()

    def pair_body(p, _):
        c0 = 2 * p
        pltpu.make_async_copy(chunk_src(c0), xb0, sem0).wait()

        @pl.when(c0 + 1 < NCHUNK)
        def _():
            pltpu.make_async_copy(chunk_src(c0 + 1), xb1, sem1).start()
        _sc_row_block(xb0, w1b, sb, c0)

        @pl.when(c0 + 1 < NCHUNK)
        def _():
            pltpu.make_async_copy(chunk_src(c0 + 1), xb1, sem1).wait()

            @pl.when(c0 + 2 < NCHUNK)
            def _():
                pltpu.make_async_copy(chunk_src(c0 + 2), xb0, sem0).start()
            _sc_row_block(xb1, w1b, sb, c0 + 1)
        return 0

    lax.fori_loop(0, (NCHUNK + 1) // 2, pair_body, 0)
    pltpu.sync_copy(sb, out_hbm.at[kb, pl.ds(n_base, RPW)])


def _head_kernel(st_ref, ss_ref, b1_ref, Wa_ref, ba_ref, Wb_ref, bb_ref,
                 Wc_ref, bc_ref, out_ref):
    big = jnp.int32(2**30)

    def take_extreme(v, sign):
        gidx = jax.lax.broadcasted_iota(jnp.int32, v.shape, 1)
        m = (jnp.max(v, axis=1, keepdims=True) if sign > 0
             else jnp.min(v, axis=1, keepdims=True))
        fi = jnp.min(jnp.where(v == m, gidx, big), axis=1, keepdims=True)
        v2 = jnp.where(gidx == fi,
                       jnp.float32(-jnp.inf) if sign > 0 else jnp.float32(jnp.inf),
                       v)
        return m, v2

    def extremes(vals):
        maxs = []
        v = vals
        for _ in range(R):
            m, v = take_extreme(v, +1)
            maxs.append(m)
        mins = []
        v = vals
        for _ in range(R):
            m, v = take_extreme(v, -1)
            mins.append(m)
        return jnp.concatenate(mins + maxs, axis=1)

    b1 = b1_ref[0]
    cat = jnp.concatenate(
        [extremes(st_ref[...] + b1), extremes(ss_ref[...] + b1)], axis=0)
    h = jnp.dot(cat, Wa_ref[:].T, preferred_element_type=jnp.float32) + ba_ref[:]
    h = jnp.dot(h, Wb_ref[:].T, preferred_element_type=jnp.float32) + bb_ref[:]
    o = jnp.dot(h, Wc_ref[:].T, preferred_element_type=jnp.float32) + bc_ref[:]
    out_ref[...] = o[:, None, :]


@jax.jit
def _chowder(x, w1, b1, Wa, ba, Wb, bb, Wc, bc):
    s_tc = pl.pallas_call(
        _score_kernel,
        grid=(T,),
        in_specs=[
            pl.BlockSpec((1, N, L), lambda b: (b, 0, 0)),
            pl.BlockSpec((L,), lambda b: (0,)),
        ],
        out_specs=pl.BlockSpec((T, N), lambda b: (0, 0)),
        out_shape=jax.ShapeDtypeStruct((T, N), jnp.float32),
        compiler_params=pltpu.CompilerParams(
            dimension_semantics=("arbitrary",),
        ),
    )(x, w1)

    sc_score = functools.partial(
        pl.kernel,
        mesh=plsc.VectorSubcoreMesh(core_axis_name="c", subcore_axis_name="s"),
        out_type=jax.ShapeDtypeStruct((K, N), jnp.float32),
        scratch_types=[
            pltpu.VMEM((CH, L), jnp.float32),
            pltpu.VMEM((CH, L), jnp.float32),
            pltpu.VMEM((L,), jnp.float32),
            pltpu.VMEM((RPW,), jnp.float32),
            pltpu.SemaphoreType.DMA,
            pltpu.SemaphoreType.DMA,
        ],
    )(_sc_score_body)
    s_sc = sc_score(x, w1)

    out = pl.pallas_call(
        _head_kernel,
        in_specs=[
            pl.BlockSpec((T, N), lambda: (0, 0)),
            pl.BlockSpec((K, N), lambda: (0, 0)),
            pl.BlockSpec((1,), lambda: (0,)),
            pl.BlockSpec((200, 2 * R), lambda: (0, 0)),
            pl.BlockSpec((200,), lambda: (0,)),
            pl.BlockSpec((100, 200), lambda: (0, 0)),
            pl.BlockSpec((100,), lambda: (0,)),
            pl.BlockSpec((C, 100), lambda: (0, 0)),
            pl.BlockSpec((C,), lambda: (0,)),
        ],
        out_specs=pl.BlockSpec((B, 1, C), lambda: (0, 0, 0)),
        out_shape=jax.ShapeDtypeStruct((B, 1, C), jnp.float32),
    )(s_tc, s_sc, b1, Wa, ba, Wb, bb, Wc, bc)
    return out


def kernel(x, w1, b1, Wa, ba, Wb, bb, Wc, bc):
    out = _chowder(x.astype(jnp.float32), w1, b1, Wa, ba, Wb, bb, Wc, bc)
    return (out, None)


# speedup vs baseline: 2.4681x; 1.2095x over previous
import jax
import jax.numpy as jnp
from jax.experimental import pallas as pl
from jax.experimental.pallas import tpu as pltpu

B, N, L, R, C = 16, 8192, 512, 5, 2


def _chowder_kernel(x_ref, w1_ref, b1_ref, Wa_ref, ba_ref, Wb_ref, bb_ref,
                    Wc_ref, bc_ref, out_ref, s_scr):
    b = pl.program_id(0)
    w = w1_ref[:].reshape(1, L)
    s = jax.lax.dot_general(w, x_ref[0], (((1,), (1,)), ((), ())),
                            preferred_element_type=jnp.float32)
    s_scr[pl.ds(b, 1), :] = s

    @pl.when(b == B - 1)
    def _finish():
        vals = s_scr[...] + b1_ref[0]
        gidx = jax.lax.broadcasted_iota(jnp.int32, (B, N), 1)
        big = jnp.int32(2**30)

        def take_extreme(v, sign):
            m = (jnp.max(v, axis=1, keepdims=True) if sign > 0
                 else jnp.min(v, axis=1, keepdims=True))
            fi = jnp.min(jnp.where(v == m, gidx, big), axis=1, keepdims=True)
            v2 = jnp.where(gidx == fi,
                           jnp.float32(-jnp.inf) if sign > 0
                           else jnp.float32(jnp.inf),
                           v)
            return m, v2

        maxs = []
        v = vals
        for _ in range(R):
            m, v = take_extreme(v, +1)
            maxs.append(m)
        mins = []
        v = vals
        for _ in range(R):
            m, v = take_extreme(v, -1)
            mins.append(m)

        cat = jnp.concatenate(mins + maxs, axis=1)
        h = jnp.dot(cat, Wa_ref[:].T,
                    preferred_element_type=jnp.float32) + ba_ref[:]
        h = jnp.dot(h, Wb_ref[:].T,
                    preferred_element_type=jnp.float32) + bb_ref[:]
        o = jnp.dot(h, Wc_ref[:].T,
                    preferred_element_type=jnp.float32) + bc_ref[:]
        out_ref[...] = o[:, None, :]


@jax.jit
def _chowder(x, w1, b1, Wa, ba, Wb, bb, Wc, bc):
    out = pl.pallas_call(
        _chowder_kernel,
        grid=(B,),
        in_specs=[
            pl.BlockSpec((1, N, L), lambda b: (b, 0, 0)),
            pl.BlockSpec((L,), lambda b: (0,)),
            pl.BlockSpec((1,), lambda b: (0,)),
            pl.BlockSpec((200, 2 * R), lambda b: (0, 0)),
            pl.BlockSpec((200,), lambda b: (0,)),
            pl.BlockSpec((100, 200), lambda b: (0, 0)),
            pl.BlockSpec((100,), lambda b: (0,)),
            pl.BlockSpec((C, 100), lambda b: (0, 0)),
            pl.BlockSpec((C,), lambda b: (0,)),
        ],
        out_specs=pl.BlockSpec((B, 1, C), lambda b: (0, 0, 0)),
        out_shape=jax.ShapeDtypeStruct((B, 1, C), jnp.float32),
        scratch_shapes=[pltpu.VMEM((B, N), jnp.float32)],
        compiler_params=pltpu.CompilerParams(
            dimension_semantics=("arbitrary",),
        ),
    )(x, w1, b1, Wa, ba, Wb, bb, Wc, bc)
    return out


def kernel(x, w1, b1, Wa, ba, Wb, bb, Wc, bc):
    out = _chowder(x.astype(jnp.float32), w1, b1, Wa, ba, Wb, bb, Wc, bc)
    return (out, None)
